# baseline (speedup 1.0000x reference)
import functools

import jax
import jax.numpy as jnp
from jax import lax
from jax.experimental import pallas as pl
from jax.experimental.pallas import tpu as pltpu
from jax.experimental.pallas import tpu_sc as plsc

N = 10000
NPAD = 10240
E = 320000
NW = 32
CHUNK = 128
CPW = 80
EPAD = NW * CPW * CHUNK
RPT = NPAD // 16
RPN = N // 16
CPO = 125
BM = 1000
GRID = N // BM

_MESH = plsc.VectorSubcoreMesh(core_axis_name="c", subcore_axis_name="s")
_SC_PARAMS = pltpu.CompilerParams(use_tc_tiling_on_sc=False)


def _fill(ref, val, d):
  v = jnp.full((16,), val, jnp.float32)

  def body(i, carry):
    for cblk in range(d // 16):
      ref[i, pl.ds(cblk * 16, 16)] = v
    return carry

  lax.fori_loop(0, CHUNK, body, 0)


def _make_sc_scatter(d, group, split=False):
  nbuf = 2 * group
  cpt = 2 * CPW if split else CPW
  ngrp = cpt // group
  assert cpt % (2 * group) == 0
  super_steps = ngrp // 2

  def body(g_hbm, src_hbm, srcp_hbm, dst_hbm, dstp_hbm, out_hbm,
           src_v, dst_v, rows_v, acc_sh, sem_g, sem_s):
    c = lax.axis_index("c")
    s = lax.axis_index("s")
    table = g_hbm.at[c] if split else g_hbm

    def load_idx(big, padarr, dest):
      @pl.when(s < 15)
      def _():
        pltpu.sync_copy(big.at[pl.ds(160 * s, 160)], dest)

      @pl.when(s == 15)
      def _():
        pltpu.sync_copy(big.at[pl.ds(2400, 100)], dest.at[pl.ds(0, 100)])
        pltpu.sync_copy(padarr, dest.at[pl.ds(100, 60)])

    load_idx(src_hbm, srcp_hbm, src_v)
    load_idx(dst_hbm, dstp_hbm, dst_v)

    def row(jj):
      return jj if split else 2 * jj + c

    def g_fire(jj, buf):
      pltpu.async_copy(table.at[src_v.at[row(jj)]], rows_v.at[pl.ds(buf * CHUNK, CHUNK)], sem_g)

    def g_wait(jj, buf):
      pltpu.make_async_copy(table.at[src_v.at[row(jj)]], rows_v.at[pl.ds(buf * CHUNK, CHUNK)],
                            sem_g).wait()

    def s_fire(jj, buf):
      pltpu.async_copy(rows_v.at[pl.ds(buf * CHUNK, CHUNK)],
                       acc_sh.at[dst_v.at[row(jj)]], sem_s, add=True)

    def s_wait(jj, buf):
      pltpu.make_async_copy(rows_v.at[pl.ds(buf * CHUNK, CHUNK)],
                            acc_sh.at[dst_v.at[row(jj)]], sem_s).wait()

    _fill(rows_v.at[pl.ds(0, CHUNK)], 0.0, d)
    for j in range(RPT // CHUNK):
      pltpu.sync_copy(rows_v.at[pl.ds(0, CHUNK)],
                      acc_sh.at[pl.ds(s * RPT + j * CHUNK, CHUNK)])
    plsc.subcore_barrier()

    for b in range(group):
      g_fire(b, b)

    def step(t, carry):
      for b in range(group):
        @pl.when(t > 0)
        def _(b=b):
          s_wait((2 * t - 1) * group + b, group + b)
        g_fire((2 * t + 1) * group + b, group + b)
      for b in range(group):
        g_wait(2 * t * group + b, b)
        s_fire(2 * t * group + b, b)
      for b in range(group):
        s_wait(2 * t * group + b, b)

        @pl.when(t < super_steps - 1)
        def _(b=b):
          g_fire((2 * t + 2) * group + b, b)
      for b in range(group):
        g_wait((2 * t + 1) * group + b, group + b)
        s_fire((2 * t + 1) * group + b, group + b)
      return carry

    lax.fori_loop(0, super_steps, step, 0)
    for b in range(group):
      s_wait((ngrp - 1) * group + b, group + b)
    plsc.subcore_barrier()

    for j in range(RPN // CPO):
      off = s * RPN + j * CPO
      pltpu.sync_copy(acc_sh.at[pl.ds(off, CPO)],
                      out_hbm.at[c, pl.ds(off, CPO)])

  return pl.kernel(
      body,
      mesh=_MESH,
      out_type=jax.ShapeDtypeStruct((2, N, d), jnp.float32),
      compiler_params=_SC_PARAMS,
      scratch_types=[
          pltpu.VMEM((2 * CPW, CHUNK), jnp.int32),
          pltpu.VMEM((2 * CPW, CHUNK), jnp.int32),
          pltpu.VMEM((nbuf * CHUNK, d), jnp.float32),
          pltpu.VMEM_SHARED((NPAD, d), jnp.float32),
          pltpu.SemaphoreType.DMA,
          pltpu.SemaphoreType.DMA,
      ],
  )


def _make_sc_hist():
  d = 16
  group = 8
  ngrp = CPW // group

  def body(dst_hbm, dstp_hbm, out_hbm, dst_v, rows_v, acc_sh, sem):
    c = lax.axis_index("c")
    s = lax.axis_index("s")

    @pl.when(s < 15)
    def _():
      pltpu.sync_copy(dst_hbm.at[pl.ds(160 * s, 160)], dst_v)

    @pl.when(s == 15)
    def _():
      pltpu.sync_copy(dst_hbm.at[pl.ds(2400, 100)], dst_v.at[pl.ds(0, 100)])
      pltpu.sync_copy(dstp_hbm, dst_v.at[pl.ds(100, 60)])

    def row(jj):
      return 2 * jj + c

    _fill(rows_v, 0.0, d)
    for j in range(RPT // CHUNK):
      pltpu.async_copy(rows_v, acc_sh.at[pl.ds(s * RPT + j * CHUNK, CHUNK)],
                       sem)
    for j in range(RPT // CHUNK):
      pltpu.make_async_copy(rows_v,
                            acc_sh.at[pl.ds(s * RPT + j * CHUNK, CHUNK)],
                            sem).wait()
    plsc.subcore_barrier()
    _fill(rows_v, 1.0, d)

    def s_fire(jj):
      pltpu.async_copy(rows_v, acc_sh.at[dst_v.at[row(jj)]], sem, add=True)

    def s_wait(jj):
      pltpu.make_async_copy(rows_v, acc_sh.at[dst_v.at[row(jj)]], sem).wait()

    for b in range(group):
      s_fire(b)

    def step(t, carry):
      for b in range(group):
        @pl.when(t < ngrp - 1)
        def _(b=b):
          s_fire((t + 1) * group + b)
        s_wait(t * group + b)
      return carry

    lax.fori_loop(0, ngrp, step, 0)
    plsc.subcore_barrier()

    for j in range(RPN // CPO):
      off = s * RPN + j * CPO
      pltpu.async_copy(acc_sh.at[pl.ds(off, CPO)],
                       out_hbm.at[c, pl.ds(off, CPO)], sem)
    for j in range(RPN // CPO):
      off = s * RPN + j * CPO
      pltpu.make_async_copy(acc_sh.at[pl.ds(off, CPO)],
                            out_hbm.at[c, pl.ds(off, CPO)], sem).wait()

  return pl.kernel(
      body,
      mesh=_MESH,
      out_type=jax.ShapeDtypeStruct((2, N, d), jnp.float32),
      compiler_params=_SC_PARAMS,
      scratch_types=[
          pltpu.VMEM((2 * CPW, CHUNK), jnp.int32),
          pltpu.VMEM((CHUNK, d), jnp.float32),
          pltpu.VMEM_SHARED((NPAD, d), jnp.float32),
          pltpu.SemaphoreType.DMA,
      ],
  )


_SC_HIST = _make_sc_hist()
_SC_SCATTER = {16: _make_sc_scatter(16, 8),
               64: _make_sc_scatter(64, 2)}
_SC_SCATTER_SPLIT128 = _make_sc_scatter(64, 2, split=True)


def _row_spec(d):
  return pl.BlockSpec((BM, d), lambda i: (i, 0))


def _full_spec(r, c):
  return pl.BlockSpec((r, c), lambda i: (0, 0))


def _pair_spec(d):
  return pl.BlockSpec((2, BM, d), lambda i: (0, i, 0))


def _dinv(dar):
  return lax.rsqrt(dar[0, :, 0:1] + dar[1, :, 0:1] + 1.0)


def _tc_first(dacc, x, w):
  din, dout = w.shape

  def f(dar, xr, wr, outr):
    dinv = _dinv(dar[...])
    outr[...] = dinv * jnp.dot(xr[...], wr[...],
                               preferred_element_type=jnp.float32)

  return pl.pallas_call(
      f,
      grid=(GRID,),
      in_specs=[_pair_spec(16), _row_spec(din), _full_spec(din, dout)],
      out_specs=_row_spec(dout),
      out_shape=jax.ShapeDtypeStruct((N, dout), jnp.float32),
  )(dacc, x, w)


def _tc_mid_out_split(dacc, a, g, brow, w):
  din = w.shape[0]

  def f(dar, ar, gr, br, wr, outr):
    dinv = _dinv(dar[...])
    t = jnp.maximum(
        dinv * (ar[0, :, :] + ar[1, :, :] + gr[...]) + br[0:1, :], 0.0)
    res = dinv * jnp.dot(t, wr[...], preferred_element_type=jnp.float32)
    outr[0, :, :] = res[:, :64]
    outr[1, :, :] = res[:, 64:]

  return pl.pallas_call(
      f,
      grid=(GRID,),
      in_specs=[_pair_spec(16), _pair_spec(din), _row_spec(din),
                _full_spec(8, din), _full_spec(din, 128)],
      out_specs=_pair_spec(64),
      out_shape=jax.ShapeDtypeStruct((2, N, 64), jnp.float32),
  )(dacc, a, g, brow, w)


def _tc_mid_in_split(dacc, a2, gs, brow, w):
  dout = w.shape[1]

  def f(dar, ar, gsr, br, wr, outr):
    dinv = _dinv(dar[...])
    tL = jnp.maximum(dinv * (ar[0, :, :] + gsr[0, :, :]) + br[0:1, :64], 0.0)
    tR = jnp.maximum(dinv * (ar[1, :, :] + gsr[1, :, :]) + br[0:1, 64:], 0.0)
    t = jnp.concatenate([tL, tR], axis=1)
    outr[...] = dinv * jnp.dot(t, wr[...], preferred_element_type=jnp.float32)

  return pl.pallas_call(
      f,
      grid=(GRID,),
      in_specs=[_pair_spec(16), _pair_spec(64), _pair_spec(64),
                _full_spec(8, 128), _full_spec(128, dout)],
      out_specs=_row_spec(dout),
      out_shape=jax.ShapeDtypeStruct((N, dout), jnp.float32),
  )(dacc, a2, gs, brow, w)


def _tc_mid(dacc, a, g, brow, w):
  din, dout = w.shape

  def f(dar, ar, gr, br, wr, outr):
    dinv = _dinv(dar[...])
    t = jnp.maximum(
        dinv * (ar[0, :, :] + ar[1, :, :] + gr[...]) + br[0:1, :], 0.0)
    outr[...] = dinv * jnp.dot(t, wr[...],
                               preferred_element_type=jnp.float32)

  return pl.pallas_call(
      f,
      grid=(GRID,),
      in_specs=[_pair_spec(16), _pair_spec(din), _row_spec(din),
                _full_spec(8, din), _full_spec(din, dout)],
      out_specs=_row_spec(dout),
      out_shape=jax.ShapeDtypeStruct((N, dout), jnp.float32),
  )(dacc, a, g, brow, w)


def _tc_final(dacc, a, g, brow):
  d = 16

  def f(dar, ar, gr, br, outr):
    dinv = _dinv(dar[...])
    outr[...] = jnp.maximum(
        dinv * (ar[0, :, :] + ar[1, :, :] + gr[...]) + br[0:1, :], 0.0)

  return pl.pallas_call(
      f,
      grid=(GRID,),
      in_specs=[_pair_spec(16), _pair_spec(d), _row_spec(d),
                _full_spec(8, d)],
      out_specs=_row_spec(d),
      out_shape=jax.ShapeDtypeStruct((N, d), jnp.float32),
  )(dacc, a, g, brow)


def kernel(x, edge_index, W1, b1, W2, b2, W3, b3, W4, b4, W5, b5):
  f32 = jnp.float32
  idx = jnp.arange(EPAD - E, dtype=jnp.int32)
  src_2d = edge_index[0].reshape(E // CHUNK, CHUNK)
  dst_2d = edge_index[1].reshape(E // CHUNK, CHUNK)
  src_p = (idx % N).reshape(-1, CHUNK)
  dst_p = (N + idx % (NPAD - N)).reshape(-1, CHUNK)

  w4p = jnp.zeros((W4.shape[0], 16), f32).at[:, :2].set(W4)
  w5p = jnp.zeros((16, 16), f32).at[:2, :1].set(W5)

  def brow(b, d):
    return jnp.broadcast_to(
        jnp.zeros((d,), f32).at[:b.shape[0]].set(b), (8, d))

  dacc = _SC_HIST(dst_2d, dst_p)

  g1 = _tc_first(dacc, x, W1)
  a = _SC_SCATTER[64](g1, src_2d, src_p, dst_2d, dst_p)
  g2 = _tc_mid_out_split(dacc, a, g1, brow(b1, 64), W2)
  a2 = _SC_SCATTER_SPLIT128(g2, src_2d, src_p, dst_2d, dst_p)
  g3 = _tc_mid_in_split(dacc, a2, g2, brow(b2, 128), W3)
  a = _SC_SCATTER[64](g3, src_2d, src_p, dst_2d, dst_p)
  g4 = _tc_mid(dacc, a, g3, brow(b3, 64), w4p)
  a = _SC_SCATTER[16](g4, src_2d, src_p, dst_2d, dst_p)
  g5 = _tc_mid(dacc, a, g4, brow(b4, 16), w5p)
  a = _SC_SCATTER[16](g5, src_2d, src_p, dst_2d, dst_p)
  out = _tc_final(dacc, a, g5, brow(b5, 16))
  return out[:, :1]


# speedup vs baseline: 1.0128x; 1.0128x over previous
import functools

import jax
import jax.numpy as jnp
from jax import lax
from jax.experimental import pallas as pl
from jax.experimental.pallas import tpu as pltpu
from jax.experimental.pallas import tpu_sc as plsc

N = 10000
NPAD = 10240
E = 320000
NW = 32
CHUNK = 128
CPW = 80
EPAD = NW * CPW * CHUNK
RPT = NPAD // 16
RPN = N // 16
CPO = 125
BM = 1000
GRID = N // BM

_MESH = plsc.VectorSubcoreMesh(core_axis_name="c", subcore_axis_name="s")
_SC_PARAMS = pltpu.CompilerParams(use_tc_tiling_on_sc=False)


def _fill(ref, val, d):
  v = jnp.full((16,), val, jnp.float32)

  def body(i, carry):
    for cblk in range(d // 16):
      ref[i, pl.ds(cblk * 16, 16)] = v
    return carry

  lax.fori_loop(0, CHUNK, body, 0)


def _make_sc_scatter(d, group, split=False):
  nbuf = 2 * group
  cpt = 2 * CPW if split else CPW
  ngrp = cpt // group
  assert cpt % (2 * group) == 0
  super_steps = ngrp // 2

  def body(g_hbm, src_hbm, dst_hbm, out_hbm, src_v, dst_v, rows_v, acc_sh,
           sem_g, sem_s):
    c = lax.axis_index("c")
    s = lax.axis_index("s")
    table = g_hbm.at[c] if split else g_hbm
    pltpu.sync_copy(src_hbm.at[s], src_v)
    pltpu.sync_copy(dst_hbm.at[s], dst_v)

    def row(jj):
      return jj if split else 2 * jj + c

    def g_fire(jj, buf):
      pltpu.async_copy(table.at[src_v.at[row(jj)]], rows_v.at[pl.ds(buf * CHUNK, CHUNK)], sem_g)

    def g_wait(jj, buf):
      pltpu.make_async_copy(table.at[src_v.at[row(jj)]], rows_v.at[pl.ds(buf * CHUNK, CHUNK)],
                            sem_g).wait()

    def s_fire(jj, buf):
      pltpu.async_copy(rows_v.at[pl.ds(buf * CHUNK, CHUNK)],
                       acc_sh.at[dst_v.at[row(jj)]], sem_s, add=True)

    def s_wait(jj, buf):
      pltpu.make_async_copy(rows_v.at[pl.ds(buf * CHUNK, CHUNK)],
                            acc_sh.at[dst_v.at[row(jj)]], sem_s).wait()

    _fill(rows_v.at[pl.ds(0, CHUNK)], 0.0, d)
    for j in range(RPT // CHUNK):
      pltpu.sync_copy(rows_v.at[pl.ds(0, CHUNK)],
                      acc_sh.at[pl.ds(s * RPT + j * CHUNK, CHUNK)])
    plsc.subcore_barrier()

    for b in range(group):
      g_fire(b, b)

    def step(t, carry):
      for b in range(group):
        @pl.when(t > 0)
        def _(b=b):
          s_wait((2 * t - 1) * group + b, group + b)
        g_fire((2 * t + 1) * group + b, group + b)
      for b in range(group):
        g_wait(2 * t * group + b, b)
        s_fire(2 * t * group + b, b)
      for b in range(group):
        s_wait(2 * t * group + b, b)

        @pl.when(t < super_steps - 1)
        def _(b=b):
          g_fire((2 * t + 2) * group + b, b)
      for b in range(group):
        g_wait((2 * t + 1) * group + b, group + b)
        s_fire((2 * t + 1) * group + b, group + b)
      return carry

    lax.fori_loop(0, super_steps, step, 0)
    for b in range(group):
      s_wait((ngrp - 1) * group + b, group + b)
    plsc.subcore_barrier()

    for j in range(RPN // CPO):
      off = s * RPN + j * CPO
      pltpu.sync_copy(acc_sh.at[pl.ds(off, CPO)],
                      out_hbm.at[c, pl.ds(off, CPO)])

  return pl.kernel(
      body,
      mesh=_MESH,
      out_type=jax.ShapeDtypeStruct((2, N, d), jnp.float32),
      compiler_params=_SC_PARAMS,
      scratch_types=[
          pltpu.VMEM((2 * CPW, CHUNK), jnp.int32),
          pltpu.VMEM((2 * CPW, CHUNK), jnp.int32),
          pltpu.VMEM((nbuf * CHUNK, d), jnp.float32),
          pltpu.VMEM_SHARED((NPAD, d), jnp.float32),
          pltpu.SemaphoreType.DMA,
          pltpu.SemaphoreType.DMA,
      ],
  )


def _make_sc_hist():
  d = 16
  group = 8
  ngrp = CPW // group

  def body(dst_hbm, out_hbm, dst_v, rows_v, acc_sh, sem):
    c = lax.axis_index("c")
    s = lax.axis_index("s")
    pltpu.sync_copy(dst_hbm.at[s], dst_v)

    def row(jj):
      return 2 * jj + c

    _fill(rows_v, 0.0, d)
    for j in range(RPT // CHUNK):
      pltpu.async_copy(rows_v, acc_sh.at[pl.ds(s * RPT + j * CHUNK, CHUNK)],
                       sem)
    for j in range(RPT // CHUNK):
      pltpu.make_async_copy(rows_v,
                            acc_sh.at[pl.ds(s * RPT + j * CHUNK, CHUNK)],
                            sem).wait()
    plsc.subcore_barrier()
    _fill(rows_v, 1.0, d)

    def s_fire(jj):
      pltpu.async_copy(rows_v, acc_sh.at[dst_v.at[row(jj)]], sem, add=True)

    def s_wait(jj):
      pltpu.make_async_copy(rows_v, acc_sh.at[dst_v.at[row(jj)]], sem).wait()

    for b in range(group):
      s_fire(b)

    def step(t, carry):
      for b in range(group):
        @pl.when(t < ngrp - 1)
        def _(b=b):
          s_fire((t + 1) * group + b)
        s_wait(t * group + b)
      return carry

    lax.fori_loop(0, ngrp, step, 0)
    plsc.subcore_barrier()

    for j in range(RPN // CPO):
      off = s * RPN + j * CPO
      pltpu.async_copy(acc_sh.at[pl.ds(off, CPO)],
                       out_hbm.at[c, pl.ds(off, CPO)], sem)
    for j in range(RPN // CPO):
      off = s * RPN + j * CPO
      pltpu.make_async_copy(acc_sh.at[pl.ds(off, CPO)],
                            out_hbm.at[c, pl.ds(off, CPO)], sem).wait()

  return pl.kernel(
      body,
      mesh=_MESH,
      out_type=jax.ShapeDtypeStruct((2, N, d), jnp.float32),
      compiler_params=_SC_PARAMS,
      scratch_types=[
          pltpu.VMEM((2 * CPW, CHUNK), jnp.int32),
          pltpu.VMEM((CHUNK, d), jnp.float32),
          pltpu.VMEM_SHARED((NPAD, d), jnp.float32),
          pltpu.SemaphoreType.DMA,
      ],
  )


_SC_HIST = _make_sc_hist()
_SC_SCATTER = {16: _make_sc_scatter(16, 8),
               64: _make_sc_scatter(64, 2)}
_SC_SCATTER_SPLIT128 = _make_sc_scatter(64, 2, split=True)


def _row_spec(d):
  return pl.BlockSpec((BM, d), lambda i: (i, 0))


def _full_spec(r, c):
  return pl.BlockSpec((r, c), lambda i: (0, 0))


def _pair_spec(d):
  return pl.BlockSpec((2, BM, d), lambda i: (0, i, 0))


def _dinv(dar):
  return lax.rsqrt(dar[0, :, 0:1] + dar[1, :, 0:1] + 1.0)


def _tc_first(dacc, x, w):
  din, dout = w.shape

  def f(dar, xr, wr, outr):
    dinv = _dinv(dar[...])
    outr[...] = dinv * jnp.dot(xr[...], wr[...],
                               preferred_element_type=jnp.float32)

  return pl.pallas_call(
      f,
      grid=(GRID,),
      in_specs=[_pair_spec(16), _row_spec(din), _full_spec(din, dout)],
      out_specs=_row_spec(dout),
      out_shape=jax.ShapeDtypeStruct((N, dout), jnp.float32),
  )(dacc, x, w)


def _tc_mid_out_split(dacc, a, g, brow, w):
  din = w.shape[0]

  def f(dar, ar, gr, br, wr, outr):
    dinv = _dinv(dar[...])
    t = jnp.maximum(
        dinv * (ar[0, :, :] + ar[1, :, :] + gr[...]) + br[0:1, :], 0.0)
    res = dinv * jnp.dot(t, wr[...], preferred_element_type=jnp.float32)
    outr[0, :, :] = res[:, :64]
    outr[1, :, :] = res[:, 64:]

  return pl.pallas_call(
      f,
      grid=(GRID,),
      in_specs=[_pair_spec(16), _pair_spec(din), _row_spec(din),
                _full_spec(8, din), _full_spec(din, 128)],
      out_specs=_pair_spec(64),
      out_shape=jax.ShapeDtypeStruct((2, N, 64), jnp.float32),
  )(dacc, a, g, brow, w)


def _tc_mid_in_split(dacc, a2, gs, brow, w):
  dout = w.shape[1]

  def f(dar, ar, gsr, br, wr, outr):
    dinv = _dinv(dar[...])
    tL = jnp.maximum(dinv * (ar[0, :, :] + gsr[0, :, :]) + br[0:1, :64], 0.0)
    tR = jnp.maximum(dinv * (ar[1, :, :] + gsr[1, :, :]) + br[0:1, 64:], 0.0)
    t = jnp.concatenate([tL, tR], axis=1)
    outr[...] = dinv * jnp.dot(t, wr[...], preferred_element_type=jnp.float32)

  return pl.pallas_call(
      f,
      grid=(GRID,),
      in_specs=[_pair_spec(16), _pair_spec(64), _pair_spec(64),
                _full_spec(8, 128), _full_spec(128, dout)],
      out_specs=_row_spec(dout),
      out_shape=jax.ShapeDtypeStruct((N, dout), jnp.float32),
  )(dacc, a2, gs, brow, w)


def _tc_mid(dacc, a, g, brow, w):
  din, dout = w.shape

  def f(dar, ar, gr, br, wr, outr):
    dinv = _dinv(dar[...])
    t = jnp.maximum(
        dinv * (ar[0, :, :] + ar[1, :, :] + gr[...]) + br[0:1, :], 0.0)
    outr[...] = dinv * jnp.dot(t, wr[...],
                               preferred_element_type=jnp.float32)

  return pl.pallas_call(
      f,
      grid=(GRID,),
      in_specs=[_pair_spec(16), _pair_spec(din), _row_spec(din),
                _full_spec(8, din), _full_spec(din, dout)],
      out_specs=_row_spec(dout),
      out_shape=jax.ShapeDtypeStruct((N, dout), jnp.float32),
  )(dacc, a, g, brow, w)


def _tc_final(dacc, a, g, brow):
  d = 16

  def f(dar, ar, gr, br, outr):
    dinv = _dinv(dar[...])
    outr[...] = jnp.maximum(
        dinv * (ar[0, :, :] + ar[1, :, :] + gr[...]) + br[0:1, :], 0.0)

  return pl.pallas_call(
      f,
      grid=(GRID,),
      in_specs=[_pair_spec(16), _pair_spec(d), _row_spec(d),
                _full_spec(8, d)],
      out_specs=_row_spec(d),
      out_shape=jax.ShapeDtypeStruct((N, d), jnp.float32),
  )(dacc, a, g, brow)


def kernel(x, edge_index, W1, b1, W2, b2, W3, b3, W4, b4, W5, b5):
  f32 = jnp.float32
  idx = jnp.arange(EPAD - E, dtype=jnp.int32)
  src_f = jnp.concatenate([edge_index[0], idx % N]).reshape(
      16, 2 * CPW, CHUNK)
  dst_f = jnp.concatenate([edge_index[1], N + idx % (NPAD - N)]).reshape(
      16, 2 * CPW, CHUNK)

  w4p = jnp.zeros((W4.shape[0], 16), f32).at[:, :2].set(W4)
  w5p = jnp.zeros((16, 16), f32).at[:2, :1].set(W5)

  def brow(b, d):
    return jnp.broadcast_to(
        jnp.zeros((d,), f32).at[:b.shape[0]].set(b), (8, d))

  dacc = _SC_HIST(dst_f)

  g1 = _tc_first(dacc, x, W1)
  a = _SC_SCATTER[64](g1, src_f, dst_f)
  g2 = _tc_mid_out_split(dacc, a, g1, brow(b1, 64), W2)
  a2 = _SC_SCATTER_SPLIT128(g2, src_f, dst_f)
  g3 = _tc_mid_in_split(dacc, a2, g2, brow(b2, 128), W3)
  a = _SC_SCATTER[64](g3, src_f, dst_f)
  g4 = _tc_mid(dacc, a, g3, brow(b3, 64), w4p)
  a = _SC_SCATTER[16](g4, src_f, dst_f)
  g5 = _tc_mid(dacc, a, g4, brow(b4, 16), w5p)
  a = _SC_SCATTER[16](g5, src_f, dst_f)
  out = _tc_final(dacc, a, g5, brow(b5, 16))
  return out[:, :1]
